# baseline (speedup 1.0000x reference)
import functools

import jax
import jax.numpy as jnp
from jax import lax
from jax.experimental import pallas as pl
from jax.experimental.pallas import tpu as pltpu
from jax.experimental.pallas import tpu_sc as plsc

NUM_NODES = 100000
DIM = 128
EPS = 1e-15
NUM_WALK_ROWS = 14080
CONTEXT = 10

ROW_BLOCK = 2000
NUM_ROW_BLOCKS = NUM_NODES // ROW_BLOCK

NW = 32
TOTAL_ROWS = 2 * NUM_WALK_ROWS
ROWS_PER_W = TOTAL_ROWS // NW
IDX_PER_W = ROWS_PER_W * CONTEXT
GROUP_ROWS = 16
IDX_PER_GROUP = GROUP_ROWS * CONTEXT
HALF_IDX = IDX_PER_GROUP // 2
N_GROUPS = ROWS_PER_W // GROUP_ROWS
PACK_WORDS = DIM // 2
D_UNROLL = 4


def _embed_body(c_ref, p0_ref, p1_ref, p2_ref, w0_ref, w1_ref, w2_ref,
                b0_ref, b1_ref, b2_ref, out_ref):
    acc = c_ref[...] + (b0_ref[...] + b1_ref[...] + b2_ref[...])
    for p_ref, w_ref in ((p0_ref, w0_ref), (p1_ref, w1_ref), (p2_ref, w2_ref)):
        acc = acc + lax.dot_general(
            p_ref[...], w_ref[...],
            dimension_numbers=(((1,), (1,)), ((), ())),
            preferred_element_type=jnp.float32)
    lo = lax.bitcast_convert_type(
        acc[:, :DIM // 2].astype(jnp.bfloat16), jnp.uint16).astype(jnp.uint32)
    hi = lax.bitcast_convert_type(
        acc[:, DIM // 2:].astype(jnp.bfloat16), jnp.uint16).astype(jnp.uint32)
    out_ref[...] = lax.bitcast_convert_type(lo | (hi << 16), jnp.float32)


def _build_embedding(c, p0, p1, p2, w0, w1, w2, b0, b1, b2):
    row_spec = pl.BlockSpec((ROW_BLOCK, DIM), lambda i: (i, 0))
    w_spec = pl.BlockSpec((DIM, DIM), lambda i: (0, 0))
    b_spec = pl.BlockSpec((1, DIM), lambda i: (0, 0))
    out_spec = pl.BlockSpec((ROW_BLOCK, DIM // 2), lambda i: (i, 0))
    return pl.pallas_call(
        _embed_body,
        grid=(NUM_ROW_BLOCKS,),
        in_specs=[row_spec, row_spec, row_spec, row_spec,
                  w_spec, w_spec, w_spec, b_spec, b_spec, b_spec],
        out_specs=out_spec,
        out_shape=jax.ShapeDtypeStruct((NUM_NODES, DIM // 2), jnp.float32),
    )(c, p0, p1, p2, w0, w1, w2, b0, b1, b2)


def _score_body(table_hbm, idx_hbm, out_hbm, idx_v, ia0, ib0, ia1, ib1,
                rows0, rows1, scores_v, sem0, sem1):
    wid = lax.axis_index("s") * 2 + lax.axis_index("c")
    pltpu.sync_copy(idx_hbm.at[pl.ds(wid * IDX_PER_W, IDX_PER_W)], idx_v)
    lane = lax.iota(jnp.int32, 16)
    lane16 = lane * 16
    row_idx = [lane * CONTEXT + s for s in range(CONTEXT)]

    def issue(g, ia, ib, rows, sem):
        for k in range(HALF_IDX // 16):
            ia[pl.ds(k * 16, 16)] = idx_v[
                pl.ds(g * IDX_PER_GROUP + k * 16, 16)]
            ib[pl.ds(k * 16, 16)] = idx_v[
                pl.ds(g * IDX_PER_GROUP + HALF_IDX + k * 16, 16)]
        pltpu.async_copy(table_hbm.at[ia], rows.at[pl.ds(0, HALF_IDX)], sem)
        pltpu.async_copy(
            table_hbm.at[ib], rows.at[pl.ds(HALF_IDX, HALF_IDX)], sem)

    def wait(ia, ib, rows, sem):
        pltpu.make_async_copy(
            table_hbm.at[ia], rows.at[pl.ds(0, HALF_IDX)], sem).wait()
        pltpu.make_async_copy(
            table_hbm.at[ib], rows.at[pl.ds(HALF_IDX, HALF_IDX)], sem).wait()

    def compute(g, rows):
        zero = jnp.zeros((32,), jnp.bfloat16)

        def d_body(it, accs):
            accs = list(accs)
            for u in range(D_UNROLL):
                d = it * D_UNROLL + u
                dvec = jnp.zeros((16,), jnp.int32) + d
                sw = plsc.bitcast(
                    plsc.load_gather(rows, [row_idx[0], dvec]), jnp.bfloat16)
                for j in range(CONTEXT - 1):
                    rw = plsc.bitcast(
                        plsc.load_gather(rows, [row_idx[j + 1], dvec]),
                        jnp.bfloat16)
                    accs[j] = accs[j] + sw * rw
            return tuple(accs)

        accs = lax.fori_loop(0, PACK_WORDS // D_UNROLL, d_body,
                             (zero,) * (CONTEXT - 1))
        base = g * (GROUP_ROWS * 16)
        for j in range(CONTEXT - 1):
            ua, ub = plsc.unpack(accs[j], format=plsc.PackFormat.INTERLEAVED)
            plsc.store_scatter(scores_v, [lane16 + (base + j)], ua + ub)

    issue(0, ia0, ib0, rows0, sem0)

    def pair_body(p, carry):
        ci = p * 2
        issue(ci + 1, ia1, ib1, rows1, sem1)
        wait(ia0, ib0, rows0, sem0)
        compute(ci, rows0)
        issue(ci + 2, ia0, ib0, rows0, sem0)
        wait(ia1, ib1, rows1, sem1)
        compute(ci + 1, rows1)
        return carry

    lax.fori_loop(0, (N_GROUPS - 1) // 2, pair_body, 0)
    wait(ia0, ib0, rows0, sem0)
    compute(N_GROUPS - 1, rows0)

    pltpu.sync_copy(
        scores_v,
        out_hbm.at[pl.ds(wid * ROWS_PER_W * 16, ROWS_PER_W * 16)])


def _compute_scores(table, idx):
    mesh = plsc.VectorSubcoreMesh(core_axis_name="c", subcore_axis_name="s")
    k = functools.partial(
        pl.kernel,
        out_type=jax.ShapeDtypeStruct((TOTAL_ROWS * 16,), jnp.float32),
        mesh=mesh,
        compiler_params=pltpu.CompilerParams(
            needs_layout_passes=False, use_tc_tiling_on_sc=False),
        scratch_types=[
            pltpu.VMEM((IDX_PER_W,), jnp.int32),
            pltpu.VMEM((HALF_IDX,), jnp.int32),
            pltpu.VMEM((HALF_IDX,), jnp.int32),
            pltpu.VMEM((HALF_IDX,), jnp.int32),
            pltpu.VMEM((HALF_IDX,), jnp.int32),
            pltpu.VMEM((IDX_PER_GROUP, PACK_WORDS), jnp.float32),
            pltpu.VMEM((IDX_PER_GROUP, PACK_WORDS), jnp.float32),
            pltpu.VMEM((ROWS_PER_W * 16,), jnp.float32),
            pltpu.SemaphoreType.DMA,
            pltpu.SemaphoreType.DMA,
        ],
    )(_score_body)
    return k(table, idx)


LOSS_ROWS = TOTAL_ROWS * 16 // 128
POS_LOSS_ROWS = LOSS_ROWS // 2


def _loss_body(s_ref, out_ref):
    x = s_ref[...]
    col = lax.broadcasted_iota(jnp.int32, (LOSS_ROWS, 128), 1)
    row = lax.broadcasted_iota(jnp.int32, (LOSS_ROWS, 128), 0)
    sig = 1.0 / (1.0 + jnp.exp(-x))
    pos_t = jnp.log(sig + EPS)
    neg_t = jnp.log(1.0 - sig + EPS)
    t = jnp.where(row < POS_LOSS_ROWS, pos_t, neg_t)
    t = jnp.where((col % 16) < (CONTEXT - 1), t, 0.0)
    denom = float(NUM_WALK_ROWS * (CONTEXT - 1))
    out_ref[0, 0] = -jnp.sum(t) / denom


def _compute_loss(scores):
    scores = scores.reshape(LOSS_ROWS, 128)
    out = pl.pallas_call(
        _loss_body,
        out_specs=pl.BlockSpec(memory_space=pltpu.SMEM),
        out_shape=jax.ShapeDtypeStruct((1, 1), jnp.float32),
    )(scores)
    return out[0, 0]


def kernel(embedding_common, embedding_private_0, embedding_private_1,
           embedding_private_2, W_0, W_1, W_2, b_0, b_1, b_2, pos_rw, neg_rw):
    table = _build_embedding(embedding_common, embedding_private_0,
                             embedding_private_1, embedding_private_2,
                             W_0, W_1, W_2, b_0.reshape(1, DIM),
                             b_1.reshape(1, DIM), b_2.reshape(1, DIM))
    idx = jnp.concatenate([pos_rw.reshape(-1), neg_rw.reshape(-1)]
                          ).astype(jnp.int32)
    scores = _compute_scores(table, idx)
    return _compute_loss(scores)


# speedup vs baseline: 1.9925x; 1.9925x over previous
import functools

import jax
import jax.numpy as jnp
from jax import lax
from jax.experimental import pallas as pl
from jax.experimental.pallas import tpu as pltpu
from jax.experimental.pallas import tpu_sc as plsc

NUM_NODES = 100000
DIM = 128
EPS = 1e-15
NUM_WALK_ROWS = 14080
CONTEXT = 10

ROW_BLOCK = 2000
NUM_ROW_BLOCKS = NUM_NODES // ROW_BLOCK

NW = 32
TOTAL_ROWS = 2 * NUM_WALK_ROWS
ROWS_PER_W = TOTAL_ROWS // NW
IDX_PER_W = ROWS_PER_W * CONTEXT
GROUP_ROWS = 16
IDX_PER_GROUP = GROUP_ROWS * CONTEXT
HALF_IDX = IDX_PER_GROUP // 2
N_GROUPS = ROWS_PER_W // GROUP_ROWS
PACK_WORDS = DIM // 2
D_UNROLL = 4


def _embed_body(c_ref, p0_ref, p1_ref, p2_ref, w0_ref, w1_ref, w2_ref,
                b0_ref, b1_ref, b2_ref, out_ref):
    acc = c_ref[...] + (b0_ref[...] + b1_ref[...] + b2_ref[...])
    for p_ref, w_ref in ((p0_ref, w0_ref), (p1_ref, w1_ref), (p2_ref, w2_ref)):
        acc = acc + lax.dot_general(
            p_ref[...], w_ref[...],
            dimension_numbers=(((1,), (1,)), ((), ())),
            preferred_element_type=jnp.float32)
    lo = lax.bitcast_convert_type(
        acc[:, :DIM // 2].astype(jnp.bfloat16), jnp.uint16).astype(jnp.uint32)
    hi = lax.bitcast_convert_type(
        acc[:, DIM // 2:].astype(jnp.bfloat16), jnp.uint16).astype(jnp.uint32)
    out_ref[...] = lax.bitcast_convert_type(lo | (hi << 16), jnp.float32)


def _build_embedding(c, p0, p1, p2, w0, w1, w2, b0, b1, b2):
    row_spec = pl.BlockSpec((ROW_BLOCK, DIM), lambda i: (i, 0))
    w_spec = pl.BlockSpec((DIM, DIM), lambda i: (0, 0))
    b_spec = pl.BlockSpec((1, DIM), lambda i: (0, 0))
    out_spec = pl.BlockSpec((ROW_BLOCK, DIM // 2), lambda i: (i, 0))
    return pl.pallas_call(
        _embed_body,
        grid=(NUM_ROW_BLOCKS,),
        in_specs=[row_spec, row_spec, row_spec, row_spec,
                  w_spec, w_spec, w_spec, b_spec, b_spec, b_spec],
        out_specs=out_spec,
        out_shape=jax.ShapeDtypeStruct((NUM_NODES, DIM // 2), jnp.float32),
    )(c, p0, p1, p2, w0, w1, w2, b0, b1, b2)


def _score_body(table_hbm, idx_hbm, out_hbm, idx_v, ia0, ib0, ia1, ib1,
                rows0, rows1, scores_v, sem0, sem1):
    wid = lax.axis_index("s") * 2 + lax.axis_index("c")
    pltpu.sync_copy(idx_hbm.at[pl.ds(wid * IDX_PER_W, IDX_PER_W)], idx_v)
    lane = lax.iota(jnp.int32, 16)
    lane16 = lane * 16
    row_idx = [lane * CONTEXT + s for s in range(CONTEXT)]

    def issue(g, ia, ib, rows, sem):
        for k in range(HALF_IDX // 16):
            ia[pl.ds(k * 16, 16)] = idx_v[
                pl.ds(g * IDX_PER_GROUP + k * 16, 16)]
            ib[pl.ds(k * 16, 16)] = idx_v[
                pl.ds(g * IDX_PER_GROUP + HALF_IDX + k * 16, 16)]
        pltpu.async_copy(table_hbm.at[ia], rows.at[pl.ds(0, HALF_IDX)], sem)
        pltpu.async_copy(
            table_hbm.at[ib], rows.at[pl.ds(HALF_IDX, HALF_IDX)], sem)

    def wait(ia, ib, rows, sem):
        pltpu.make_async_copy(
            table_hbm.at[ia], rows.at[pl.ds(0, HALF_IDX)], sem).wait()
        pltpu.make_async_copy(
            table_hbm.at[ib], rows.at[pl.ds(HALF_IDX, HALF_IDX)], sem).wait()

    def compute(g, rows):
        zero = jnp.zeros((32,), jnp.bfloat16)

        def d_body(it, accs):
            accs = list(accs)
            for u in range(D_UNROLL):
                d = it * D_UNROLL + u
                dvec = (lane + d) & (PACK_WORDS - 1)
                sw = plsc.bitcast(
                    plsc.load_gather(rows, [row_idx[0], dvec]), jnp.bfloat16)
                for j in range(CONTEXT - 1):
                    rw = plsc.bitcast(
                        plsc.load_gather(rows, [row_idx[j + 1], dvec]),
                        jnp.bfloat16)
                    accs[j] = accs[j] + sw * rw
            return tuple(accs)

        accs = lax.fori_loop(0, PACK_WORDS // D_UNROLL, d_body,
                             (zero,) * (CONTEXT - 1))
        base = g * (GROUP_ROWS * 16)
        for j in range(CONTEXT - 1):
            ua, ub = plsc.unpack(accs[j], format=plsc.PackFormat.INTERLEAVED)
            plsc.store_scatter(scores_v, [lane16 + (base + j)], ua + ub)

    issue(0, ia0, ib0, rows0, sem0)

    def pair_body(p, carry):
        ci = p * 2
        issue(ci + 1, ia1, ib1, rows1, sem1)
        wait(ia0, ib0, rows0, sem0)
        compute(ci, rows0)
        issue(ci + 2, ia0, ib0, rows0, sem0)
        wait(ia1, ib1, rows1, sem1)
        compute(ci + 1, rows1)
        return carry

    lax.fori_loop(0, (N_GROUPS - 1) // 2, pair_body, 0)
    wait(ia0, ib0, rows0, sem0)
    compute(N_GROUPS - 1, rows0)

    pltpu.sync_copy(
        scores_v,
        out_hbm.at[pl.ds(wid * ROWS_PER_W * 16, ROWS_PER_W * 16)])


def _compute_scores(table, idx):
    mesh = plsc.VectorSubcoreMesh(core_axis_name="c", subcore_axis_name="s")
    k = functools.partial(
        pl.kernel,
        out_type=jax.ShapeDtypeStruct((TOTAL_ROWS * 16,), jnp.float32),
        mesh=mesh,
        compiler_params=pltpu.CompilerParams(
            needs_layout_passes=False, use_tc_tiling_on_sc=False),
        scratch_types=[
            pltpu.VMEM((IDX_PER_W,), jnp.int32),
            pltpu.VMEM((HALF_IDX,), jnp.int32),
            pltpu.VMEM((HALF_IDX,), jnp.int32),
            pltpu.VMEM((HALF_IDX,), jnp.int32),
            pltpu.VMEM((HALF_IDX,), jnp.int32),
            pltpu.VMEM((IDX_PER_GROUP, PACK_WORDS), jnp.float32),
            pltpu.VMEM((IDX_PER_GROUP, PACK_WORDS), jnp.float32),
            pltpu.VMEM((ROWS_PER_W * 16,), jnp.float32),
            pltpu.SemaphoreType.DMA,
            pltpu.SemaphoreType.DMA,
        ],
    )(_score_body)
    return k(table, idx)


LOSS_ROWS = TOTAL_ROWS * 16 // 128
POS_LOSS_ROWS = LOSS_ROWS // 2


def _loss_body(s_ref, out_ref):
    x = s_ref[...]
    col = lax.broadcasted_iota(jnp.int32, (LOSS_ROWS, 128), 1)
    row = lax.broadcasted_iota(jnp.int32, (LOSS_ROWS, 128), 0)
    sig = 1.0 / (1.0 + jnp.exp(-x))
    pos_t = jnp.log(sig + EPS)
    neg_t = jnp.log(1.0 - sig + EPS)
    t = jnp.where(row < POS_LOSS_ROWS, pos_t, neg_t)
    t = jnp.where((col % 16) < (CONTEXT - 1), t, 0.0)
    denom = float(NUM_WALK_ROWS * (CONTEXT - 1))
    out_ref[0, 0] = -jnp.sum(t) / denom


def _compute_loss(scores):
    scores = scores.reshape(LOSS_ROWS, 128)
    out = pl.pallas_call(
        _loss_body,
        out_specs=pl.BlockSpec(memory_space=pltpu.SMEM),
        out_shape=jax.ShapeDtypeStruct((1, 1), jnp.float32),
    )(scores)
    return out[0, 0]


def kernel(embedding_common, embedding_private_0, embedding_private_1,
           embedding_private_2, W_0, W_1, W_2, b_0, b_1, b_2, pos_rw, neg_rw):
    table = _build_embedding(embedding_common, embedding_private_0,
                             embedding_private_1, embedding_private_2,
                             W_0, W_1, W_2, b_0.reshape(1, DIM),
                             b_1.reshape(1, DIM), b_2.reshape(1, DIM))
    idx = jnp.concatenate([pos_rw.reshape(-1), neg_rw.reshape(-1)]
                          ).astype(jnp.int32)
    scores = _compute_scores(table, idx)
    return _compute_loss(scores)


# speedup vs baseline: 2.4571x; 1.2331x over previous
import functools

import jax
import jax.numpy as jnp
from jax import lax
from jax.experimental import pallas as pl
from jax.experimental.pallas import tpu as pltpu
from jax.experimental.pallas import tpu_sc as plsc

NUM_NODES = 100000
DIM = 128
EPS = 1e-15
NUM_WALK_ROWS = 14080
CONTEXT = 10

ROW_BLOCK = 2000
NUM_ROW_BLOCKS = NUM_NODES // ROW_BLOCK

NW = 32
TOTAL_ROWS = 2 * NUM_WALK_ROWS
ROWS_PER_W = TOTAL_ROWS // NW
IDX_PER_W = ROWS_PER_W * CONTEXT
GROUP_ROWS = 16
IDX_PER_GROUP = GROUP_ROWS * CONTEXT
HALF_IDX = IDX_PER_GROUP // 2
N_GROUPS = ROWS_PER_W // GROUP_ROWS
PACK_WORDS = DIM // 2
D_UNROLL = 4


def _embed_body(c_ref, p0_ref, p1_ref, p2_ref, w0_ref, w1_ref, w2_ref,
                b0_ref, b1_ref, b2_ref, out_ref):
    acc = c_ref[...] + (b0_ref[...] + b1_ref[...] + b2_ref[...])
    for p_ref, w_ref in ((p0_ref, w0_ref), (p1_ref, w1_ref), (p2_ref, w2_ref)):
        acc = acc + lax.dot_general(
            p_ref[...], w_ref[...],
            dimension_numbers=(((1,), (1,)), ((), ())),
            preferred_element_type=jnp.float32)
    lo = lax.bitcast_convert_type(
        acc[:, :DIM // 2].astype(jnp.bfloat16), jnp.uint16).astype(jnp.uint32)
    hi = lax.bitcast_convert_type(
        acc[:, DIM // 2:].astype(jnp.bfloat16), jnp.uint16).astype(jnp.uint32)
    packed = lax.bitcast_convert_type(lo | (hi << 16), jnp.float32)
    out_ref[...] = jnp.concatenate(
        [packed[:ROW_BLOCK // 2], packed[ROW_BLOCK // 2:]], axis=1)


def _build_embedding(c, p0, p1, p2, w0, w1, w2, b0, b1, b2):
    row_spec = pl.BlockSpec((ROW_BLOCK, DIM), lambda i: (i, 0))
    w_spec = pl.BlockSpec((DIM, DIM), lambda i: (0, 0))
    b_spec = pl.BlockSpec((1, DIM), lambda i: (0, 0))
    out_spec = pl.BlockSpec((ROW_BLOCK // 2, DIM), lambda i: (i, 0))
    return pl.pallas_call(
        _embed_body,
        grid=(NUM_ROW_BLOCKS,),
        in_specs=[row_spec, row_spec, row_spec, row_spec,
                  w_spec, w_spec, w_spec, b_spec, b_spec, b_spec],
        out_specs=out_spec,
        out_shape=jax.ShapeDtypeStruct((NUM_NODES // 2, DIM), jnp.float32),
    )(c, p0, p1, p2, w0, w1, w2, b0, b1, b2)


def _score_body(table_hbm, idx_hbm, out_hbm, idx_v, ia0, ib0, ia1, ib1,
                rows0, rows1, scores_v, sem0, sem1):
    wid = lax.axis_index("s") * 2 + lax.axis_index("c")
    pltpu.sync_copy(idx_hbm.at[pl.ds(wid * IDX_PER_W, IDX_PER_W)], idx_v)
    lane = lax.iota(jnp.int32, 16)
    lane16 = lane * 16
    row_idx = [lane * CONTEXT + s for s in range(CONTEXT)]

    def issue(g, ia, ib, rows, sem):
        for k in range(HALF_IDX // 16):
            ia[pl.ds(k * 16, 16)] = idx_v[
                pl.ds(g * IDX_PER_GROUP + k * 16, 16)]
            ib[pl.ds(k * 16, 16)] = idx_v[
                pl.ds(g * IDX_PER_GROUP + HALF_IDX + k * 16, 16)]
        pltpu.async_copy(table_hbm.at[ia], rows.at[pl.ds(0, HALF_IDX)], sem)
        pltpu.async_copy(
            table_hbm.at[ib], rows.at[pl.ds(HALF_IDX, HALF_IDX)], sem)

    def wait(ia, ib, rows, sem):
        pltpu.make_async_copy(
            table_hbm.at[ia], rows.at[pl.ds(0, HALF_IDX)], sem).wait()
        pltpu.make_async_copy(
            table_hbm.at[ib], rows.at[pl.ds(HALF_IDX, HALF_IDX)], sem).wait()

    def compute(g, rows):
        zero = jnp.zeros((32,), jnp.bfloat16)

        def d_body(it, accs):
            accs = list(accs)
            for u in range(D_UNROLL):
                d = it * D_UNROLL + u
                dvec = (lane + d) & (PACK_WORDS - 1)
                sw = plsc.bitcast(
                    plsc.load_gather(rows, [row_idx[0], dvec]), jnp.bfloat16)
                for j in range(CONTEXT - 1):
                    rw = plsc.bitcast(
                        plsc.load_gather(rows, [row_idx[j + 1], dvec]),
                        jnp.bfloat16)
                    accs[j] = accs[j] + sw * rw
            return tuple(accs)

        accs = lax.fori_loop(0, PACK_WORDS // D_UNROLL, d_body,
                             (zero,) * (CONTEXT - 1))
        base = g * (GROUP_ROWS * 16)
        for j in range(CONTEXT - 1):
            ua, ub = plsc.unpack(accs[j], format=plsc.PackFormat.INTERLEAVED)
            plsc.store_scatter(scores_v, [lane16 + (base + j)], ua + ub)

    issue(0, ia0, ib0, rows0, sem0)

    def pair_body(p, carry):
        ci = p * 2
        issue(ci + 1, ia1, ib1, rows1, sem1)
        wait(ia0, ib0, rows0, sem0)
        compute(ci, rows0)
        issue(ci + 2, ia0, ib0, rows0, sem0)
        wait(ia1, ib1, rows1, sem1)
        compute(ci + 1, rows1)
        return carry

    lax.fori_loop(0, (N_GROUPS - 1) // 2, pair_body, 0)
    wait(ia0, ib0, rows0, sem0)
    compute(N_GROUPS - 1, rows0)

    pltpu.sync_copy(
        scores_v,
        out_hbm.at[pl.ds(wid * ROWS_PER_W * 16, ROWS_PER_W * 16)])


def _compute_scores(table, idx):
    mesh = plsc.VectorSubcoreMesh(core_axis_name="c", subcore_axis_name="s")
    k = functools.partial(
        pl.kernel,
        out_type=jax.ShapeDtypeStruct((TOTAL_ROWS * 16,), jnp.float32),
        mesh=mesh,
        compiler_params=pltpu.CompilerParams(
            needs_layout_passes=False, use_tc_tiling_on_sc=False),
        scratch_types=[
            pltpu.VMEM((IDX_PER_W,), jnp.int32),
            pltpu.VMEM((HALF_IDX,), jnp.int32),
            pltpu.VMEM((HALF_IDX,), jnp.int32),
            pltpu.VMEM((HALF_IDX,), jnp.int32),
            pltpu.VMEM((HALF_IDX,), jnp.int32),
            pltpu.VMEM((IDX_PER_GROUP, PACK_WORDS), jnp.float32),
            pltpu.VMEM((IDX_PER_GROUP, PACK_WORDS), jnp.float32),
            pltpu.VMEM((ROWS_PER_W * 16,), jnp.float32),
            pltpu.SemaphoreType.DMA,
            pltpu.SemaphoreType.DMA,
        ],
    )(_score_body)
    return k(table, idx)


LOSS_ROWS = TOTAL_ROWS * 16 // 128
POS_LOSS_ROWS = LOSS_ROWS // 2


def _loss_body(s_ref, out_ref):
    x = s_ref[...]
    col = lax.broadcasted_iota(jnp.int32, (LOSS_ROWS, 128), 1)
    row = lax.broadcasted_iota(jnp.int32, (LOSS_ROWS, 128), 0)
    sig = 1.0 / (1.0 + jnp.exp(-x))
    pos_t = jnp.log(sig + EPS)
    neg_t = jnp.log(1.0 - sig + EPS)
    t = jnp.where(row < POS_LOSS_ROWS, pos_t, neg_t)
    t = jnp.where((col % 16) < (CONTEXT - 1), t, 0.0)
    denom = float(NUM_WALK_ROWS * (CONTEXT - 1))
    out_ref[0, 0] = -jnp.sum(t) / denom


def _compute_loss(scores):
    scores = scores.reshape(LOSS_ROWS, 128)
    out = pl.pallas_call(
        _loss_body,
        out_specs=pl.BlockSpec(memory_space=pltpu.SMEM),
        out_shape=jax.ShapeDtypeStruct((1, 1), jnp.float32),
    )(scores)
    return out[0, 0]


def kernel(embedding_common, embedding_private_0, embedding_private_1,
           embedding_private_2, W_0, W_1, W_2, b_0, b_1, b_2, pos_rw, neg_rw):
    table = _build_embedding(embedding_common, embedding_private_0,
                             embedding_private_1, embedding_private_2,
                             W_0, W_1, W_2, b_0.reshape(1, DIM),
                             b_1.reshape(1, DIM), b_2.reshape(1, DIM))
    idx = jnp.concatenate([pos_rw.reshape(-1), neg_rw.reshape(-1)]
                          ).astype(jnp.int32)
    loc = idx % (2 * (ROW_BLOCK // 2))
    idx = (idx - loc) + (loc % (ROW_BLOCK // 2)) * 2 + loc // (ROW_BLOCK // 2)
    scores = _compute_scores(table.reshape(NUM_NODES, DIM // 2), idx)
    return _compute_loss(scores)


# speedup vs baseline: 2.7401x; 1.1152x over previous
import functools

import jax
import jax.numpy as jnp
from jax import lax
from jax.experimental import pallas as pl
from jax.experimental.pallas import tpu as pltpu
from jax.experimental.pallas import tpu_sc as plsc

NUM_NODES = 100000
DIM = 128
EPS = 1e-15
NUM_WALK_ROWS = 14080
CONTEXT = 10

ROW_BLOCK = 2000
NUM_ROW_BLOCKS = NUM_NODES // ROW_BLOCK

NW = 32
TOTAL_ROWS = 2 * NUM_WALK_ROWS
ROWS_PER_W = TOTAL_ROWS // NW
IDX_PER_W = ROWS_PER_W * CONTEXT
GROUP_ROWS = 16
IDX_PER_GROUP = GROUP_ROWS * CONTEXT
HALF_IDX = IDX_PER_GROUP // 2
N_GROUPS = ROWS_PER_W // GROUP_ROWS
PACK_WORDS = DIM // 2
D_UNROLL = 4


def _embed_body(c_ref, p0_ref, p1_ref, p2_ref, w0_ref, w1_ref, w2_ref,
                b0_ref, b1_ref, b2_ref, out_ref):
    acc = c_ref[...] + (b0_ref[...] + b1_ref[...] + b2_ref[...])
    for p_ref, w_ref in ((p0_ref, w0_ref), (p1_ref, w1_ref), (p2_ref, w2_ref)):
        acc = acc + lax.dot_general(
            p_ref[...], w_ref[...],
            dimension_numbers=(((1,), (1,)), ((), ())),
            preferred_element_type=jnp.float32)
    lo = lax.bitcast_convert_type(
        acc[:, :DIM // 2].astype(jnp.bfloat16), jnp.uint16).astype(jnp.uint32)
    hi = lax.bitcast_convert_type(
        acc[:, DIM // 2:].astype(jnp.bfloat16), jnp.uint16).astype(jnp.uint32)
    packed = lax.bitcast_convert_type(lo | (hi << 16), jnp.float32)
    out_ref[...] = jnp.concatenate(
        [packed[:ROW_BLOCK // 2], packed[ROW_BLOCK // 2:]], axis=1)


def _build_embedding(c, p0, p1, p2, w0, w1, w2, b0, b1, b2):
    row_spec = pl.BlockSpec((ROW_BLOCK, DIM), lambda i: (i, 0))
    w_spec = pl.BlockSpec((DIM, DIM), lambda i: (0, 0))
    b_spec = pl.BlockSpec((1, DIM), lambda i: (0, 0))
    out_spec = pl.BlockSpec((ROW_BLOCK // 2, DIM), lambda i: (i, 0))
    return pl.pallas_call(
        _embed_body,
        grid=(NUM_ROW_BLOCKS,),
        in_specs=[row_spec, row_spec, row_spec, row_spec,
                  w_spec, w_spec, w_spec, b_spec, b_spec, b_spec],
        out_specs=out_spec,
        out_shape=jax.ShapeDtypeStruct((NUM_NODES // 2, DIM), jnp.float32),
    )(c, p0, p1, p2, w0, w1, w2, b0, b1, b2)


def _score_body(table_hbm, idx_hbm, out_hbm, idx_v, ia0, ib0, ia1, ib1,
                rows0, rows1, scores_v, sem0, sem1):
    wid = lax.axis_index("s") * 2 + lax.axis_index("c")
    part = wid // 16
    rbase = (wid % 16) * ROWS_PER_W
    for s in range(CONTEXT):
        pltpu.sync_copy(
            idx_hbm.at[pl.ds(part * (NUM_WALK_ROWS * CONTEXT)
                             + s * NUM_WALK_ROWS + rbase, ROWS_PER_W)],
            idx_v.at[pl.ds(s * ROWS_PER_W, ROWS_PER_W)])
    lane = lax.iota(jnp.int32, 16)
    lane16 = lane * 16
    row_idx = [s * GROUP_ROWS + lane for s in range(CONTEXT)]

    def issue(g, ia, ib, rows, sem):
        for k in range(HALF_IDX // 16):
            ia[pl.ds(k * 16, 16)] = idx_v[
                pl.ds(k * ROWS_PER_W + g * GROUP_ROWS, 16)]
            ib[pl.ds(k * 16, 16)] = idx_v[
                pl.ds((k + 5) * ROWS_PER_W + g * GROUP_ROWS, 16)]
        pltpu.async_copy(table_hbm.at[ia], rows.at[pl.ds(0, HALF_IDX)], sem)
        pltpu.async_copy(
            table_hbm.at[ib], rows.at[pl.ds(HALF_IDX, HALF_IDX)], sem)

    def wait(ia, ib, rows, sem):
        pltpu.make_async_copy(
            table_hbm.at[ia], rows.at[pl.ds(0, HALF_IDX)], sem).wait()
        pltpu.make_async_copy(
            table_hbm.at[ib], rows.at[pl.ds(HALF_IDX, HALF_IDX)], sem).wait()

    def compute(g, rows):
        zero = jnp.zeros((32,), jnp.bfloat16)

        def d_body(it, accs):
            accs = list(accs)
            for u in range(D_UNROLL):
                d = it * D_UNROLL + u
                dvec = (lane + d) & (PACK_WORDS - 1)
                sw = plsc.bitcast(
                    plsc.load_gather(rows, [row_idx[0], dvec]), jnp.bfloat16)
                for j in range(CONTEXT - 1):
                    rw = plsc.bitcast(
                        plsc.load_gather(rows, [row_idx[j + 1], dvec]),
                        jnp.bfloat16)
                    accs[j] = accs[j] + sw * rw
            return tuple(accs)

        accs = lax.fori_loop(0, PACK_WORDS // D_UNROLL, d_body,
                             (zero,) * (CONTEXT - 1))
        base = g * (GROUP_ROWS * 16)
        for j in range(CONTEXT - 1):
            ua, ub = plsc.unpack(accs[j], format=plsc.PackFormat.INTERLEAVED)
            plsc.store_scatter(scores_v, [lane16 + (base + j)], ua + ub)

    issue(0, ia0, ib0, rows0, sem0)

    def pair_body(p, carry):
        ci = p * 2
        issue(ci + 1, ia1, ib1, rows1, sem1)
        wait(ia0, ib0, rows0, sem0)
        compute(ci, rows0)
        issue(ci + 2, ia0, ib0, rows0, sem0)
        wait(ia1, ib1, rows1, sem1)
        compute(ci + 1, rows1)
        return carry

    lax.fori_loop(0, (N_GROUPS - 1) // 2, pair_body, 0)
    wait(ia0, ib0, rows0, sem0)
    compute(N_GROUPS - 1, rows0)

    pltpu.sync_copy(
        scores_v,
        out_hbm.at[pl.ds(wid * ROWS_PER_W * 16, ROWS_PER_W * 16)])


def _compute_scores(table, idx):
    mesh = plsc.VectorSubcoreMesh(core_axis_name="c", subcore_axis_name="s")
    k = functools.partial(
        pl.kernel,
        out_type=jax.ShapeDtypeStruct((TOTAL_ROWS * 16,), jnp.float32),
        mesh=mesh,
        compiler_params=pltpu.CompilerParams(
            needs_layout_passes=False, use_tc_tiling_on_sc=False),
        scratch_types=[
            pltpu.VMEM((IDX_PER_W,), jnp.int32),
            pltpu.VMEM((HALF_IDX,), jnp.int32),
            pltpu.VMEM((HALF_IDX,), jnp.int32),
            pltpu.VMEM((HALF_IDX,), jnp.int32),
            pltpu.VMEM((HALF_IDX,), jnp.int32),
            pltpu.VMEM((IDX_PER_GROUP, PACK_WORDS), jnp.float32),
            pltpu.VMEM((IDX_PER_GROUP, PACK_WORDS), jnp.float32),
            pltpu.VMEM((ROWS_PER_W * 16,), jnp.float32),
            pltpu.SemaphoreType.DMA,
            pltpu.SemaphoreType.DMA,
        ],
    )(_score_body)
    return k(table, idx)


LOSS_ROWS = TOTAL_ROWS * 16 // 128
POS_LOSS_ROWS = LOSS_ROWS // 2


def _loss_body(s_ref, out_ref):
    x = s_ref[...]
    col = lax.broadcasted_iota(jnp.int32, (LOSS_ROWS, 128), 1)
    row = lax.broadcasted_iota(jnp.int32, (LOSS_ROWS, 128), 0)
    sig = 1.0 / (1.0 + jnp.exp(-x))
    pos_t = jnp.log(sig + EPS)
    neg_t = jnp.log(1.0 - sig + EPS)
    t = jnp.where(row < POS_LOSS_ROWS, pos_t, neg_t)
    t = jnp.where((col % 16) < (CONTEXT - 1), t, 0.0)
    denom = float(NUM_WALK_ROWS * (CONTEXT - 1))
    out_ref[0, 0] = -jnp.sum(t) / denom


def _compute_loss(scores):
    scores = scores.reshape(LOSS_ROWS, 128)
    out = pl.pallas_call(
        _loss_body,
        out_specs=pl.BlockSpec(memory_space=pltpu.SMEM),
        out_shape=jax.ShapeDtypeStruct((1, 1), jnp.float32),
    )(scores)
    return out[0, 0]


def kernel(embedding_common, embedding_private_0, embedding_private_1,
           embedding_private_2, W_0, W_1, W_2, b_0, b_1, b_2, pos_rw, neg_rw):
    table = _build_embedding(embedding_common, embedding_private_0,
                             embedding_private_1, embedding_private_2,
                             W_0, W_1, W_2, b_0.reshape(1, DIM),
                             b_1.reshape(1, DIM), b_2.reshape(1, DIM))
    idx = jnp.concatenate([pos_rw.T.reshape(-1), neg_rw.T.reshape(-1)]
                          ).astype(jnp.int32)
    loc = idx % (2 * (ROW_BLOCK // 2))
    idx = (idx - loc) + (loc % (ROW_BLOCK // 2)) * 2 + loc // (ROW_BLOCK // 2)
    scores = _compute_scores(table.reshape(NUM_NODES, DIM // 2), idx)
    return _compute_loss(scores)


# speedup vs baseline: 2.9913x; 1.0917x over previous
import functools

import jax
import jax.numpy as jnp
from jax import lax
from jax.experimental import pallas as pl
from jax.experimental.pallas import tpu as pltpu
from jax.experimental.pallas import tpu_sc as plsc

NUM_NODES = 100000
DIM = 128
EPS = 1e-15
NUM_WALK_ROWS = 14080
CONTEXT = 10

ROW_BLOCK = 4000
NUM_ROW_BLOCKS = NUM_NODES // ROW_BLOCK

NW = 32
TOTAL_ROWS = 2 * NUM_WALK_ROWS
ROWS_PER_W = TOTAL_ROWS // NW
IDX_PER_W = ROWS_PER_W * CONTEXT
GROUP_ROWS = 16
IDX_PER_GROUP = GROUP_ROWS * CONTEXT
HALF_IDX = IDX_PER_GROUP // 2
N_GROUPS = ROWS_PER_W // GROUP_ROWS
PACK_WORDS = DIM // 2
D_UNROLL = 4


def _embed_body(c_ref, p0_ref, p1_ref, p2_ref, w0_ref, w1_ref, w2_ref,
                b0_ref, b1_ref, b2_ref, out_ref):
    acc = c_ref[...] + (b0_ref[...] + b1_ref[...] + b2_ref[...])
    for p_ref, w_ref in ((p0_ref, w0_ref), (p1_ref, w1_ref), (p2_ref, w2_ref)):
        acc = acc + lax.dot_general(
            p_ref[...], w_ref[...],
            dimension_numbers=(((1,), (1,)), ((), ())),
            preferred_element_type=jnp.float32)
    lo = lax.bitcast_convert_type(
        acc[:, :DIM // 2].astype(jnp.bfloat16), jnp.uint16).astype(jnp.uint32)
    hi = lax.bitcast_convert_type(
        acc[:, DIM // 2:].astype(jnp.bfloat16), jnp.uint16).astype(jnp.uint32)
    packed = lax.bitcast_convert_type(lo | (hi << 16), jnp.float32)
    out_ref[...] = jnp.concatenate(
        [packed[:ROW_BLOCK // 2], packed[ROW_BLOCK // 2:]], axis=1)


def _build_embedding(c, p0, p1, p2, w0, w1, w2, b0, b1, b2):
    row_spec = pl.BlockSpec((ROW_BLOCK, DIM), lambda i: (i, 0))
    w_spec = pl.BlockSpec((DIM, DIM), lambda i: (0, 0))
    b_spec = pl.BlockSpec((1, DIM), lambda i: (0, 0))
    out_spec = pl.BlockSpec((ROW_BLOCK // 2, DIM), lambda i: (i, 0))
    return pl.pallas_call(
        _embed_body,
        grid=(NUM_ROW_BLOCKS,),
        in_specs=[row_spec, row_spec, row_spec, row_spec,
                  w_spec, w_spec, w_spec, b_spec, b_spec, b_spec],
        out_specs=out_spec,
        out_shape=jax.ShapeDtypeStruct((NUM_NODES // 2, DIM), jnp.float32),
    )(c, p0, p1, p2, w0, w1, w2, b0, b1, b2)


def _score_body(table_hbm, idx_hbm, out_hbm, idx_v, ia0, ib0, ia1, ib1,
                rows0, rows1, scores_v, sem0, sem1):
    wid = lax.axis_index("s") * 2 + lax.axis_index("c")
    part = wid // 16
    rbase = (wid % 16) * ROWS_PER_W
    for s in range(CONTEXT):
        pltpu.sync_copy(
            idx_hbm.at[pl.ds(part * (NUM_WALK_ROWS * CONTEXT)
                             + s * NUM_WALK_ROWS + rbase, ROWS_PER_W)],
            idx_v.at[pl.ds(s * ROWS_PER_W, ROWS_PER_W)])
    lane = lax.iota(jnp.int32, 16)
    lane16 = lane * 16
    row_idx = [s * GROUP_ROWS + lane for s in range(CONTEXT)]

    def issue(g, ia, ib, rows, sem):
        for k in range(HALF_IDX // 16):
            ia[pl.ds(k * 16, 16)] = idx_v[
                pl.ds(k * ROWS_PER_W + g * GROUP_ROWS, 16)]
            ib[pl.ds(k * 16, 16)] = idx_v[
                pl.ds((k + 5) * ROWS_PER_W + g * GROUP_ROWS, 16)]
        pltpu.async_copy(table_hbm.at[ia], rows.at[pl.ds(0, HALF_IDX)], sem)
        pltpu.async_copy(
            table_hbm.at[ib], rows.at[pl.ds(HALF_IDX, HALF_IDX)], sem)

    def wait(ia, ib, rows, sem):
        pltpu.make_async_copy(
            table_hbm.at[ia], rows.at[pl.ds(0, HALF_IDX)], sem).wait()
        pltpu.make_async_copy(
            table_hbm.at[ib], rows.at[pl.ds(HALF_IDX, HALF_IDX)], sem).wait()

    def compute(g, rows):
        zero = jnp.zeros((32,), jnp.bfloat16)

        def d_body(it, accs):
            accs = list(accs)
            for u in range(D_UNROLL):
                d = it * D_UNROLL + u
                dvec = (lane + d) & (PACK_WORDS - 1)
                sw = plsc.bitcast(
                    plsc.load_gather(rows, [row_idx[0], dvec]), jnp.bfloat16)
                for j in range(CONTEXT - 1):
                    rw = plsc.bitcast(
                        plsc.load_gather(rows, [row_idx[j + 1], dvec]),
                        jnp.bfloat16)
                    accs[j] = accs[j] + sw * rw
            return tuple(accs)

        accs = lax.fori_loop(0, PACK_WORDS // D_UNROLL, d_body,
                             (zero,) * (CONTEXT - 1))
        base = g * (GROUP_ROWS * 16)
        for j in range(CONTEXT - 1):
            ua, ub = plsc.unpack(accs[j], format=plsc.PackFormat.INTERLEAVED)
            plsc.store_scatter(scores_v, [lane16 + (base + j)], ua + ub)

    issue(0, ia0, ib0, rows0, sem0)

    def pair_body(p, carry):
        ci = p * 2
        issue(ci + 1, ia1, ib1, rows1, sem1)
        wait(ia0, ib0, rows0, sem0)
        compute(ci, rows0)
        issue(ci + 2, ia0, ib0, rows0, sem0)
        wait(ia1, ib1, rows1, sem1)
        compute(ci + 1, rows1)
        return carry

    lax.fori_loop(0, (N_GROUPS - 1) // 2, pair_body, 0)
    wait(ia0, ib0, rows0, sem0)
    compute(N_GROUPS - 1, rows0)

    pltpu.sync_copy(
        scores_v,
        out_hbm.at[pl.ds(wid * ROWS_PER_W * 16, ROWS_PER_W * 16)])


def _compute_scores(table, idx):
    mesh = plsc.VectorSubcoreMesh(core_axis_name="c", subcore_axis_name="s")
    k = functools.partial(
        pl.kernel,
        out_type=jax.ShapeDtypeStruct((TOTAL_ROWS * 16,), jnp.float32),
        mesh=mesh,
        compiler_params=pltpu.CompilerParams(
            needs_layout_passes=False, use_tc_tiling_on_sc=False),
        scratch_types=[
            pltpu.VMEM((IDX_PER_W,), jnp.int32),
            pltpu.VMEM((HALF_IDX,), jnp.int32),
            pltpu.VMEM((HALF_IDX,), jnp.int32),
            pltpu.VMEM((HALF_IDX,), jnp.int32),
            pltpu.VMEM((HALF_IDX,), jnp.int32),
            pltpu.VMEM((IDX_PER_GROUP, PACK_WORDS), jnp.float32),
            pltpu.VMEM((IDX_PER_GROUP, PACK_WORDS), jnp.float32),
            pltpu.VMEM((ROWS_PER_W * 16,), jnp.float32),
            pltpu.SemaphoreType.DMA,
            pltpu.SemaphoreType.DMA,
        ],
    )(_score_body)
    return k(table, idx)


LOSS_ROWS = TOTAL_ROWS * 16 // 128
POS_LOSS_ROWS = LOSS_ROWS // 2


def _loss_body(s_ref, out_ref):
    x = s_ref[...]
    col = lax.broadcasted_iota(jnp.int32, (LOSS_ROWS, 128), 1)
    row = lax.broadcasted_iota(jnp.int32, (LOSS_ROWS, 128), 0)
    sig = 1.0 / (1.0 + jnp.exp(-x))
    pos_t = jnp.log(sig + EPS)
    neg_t = jnp.log(1.0 - sig + EPS)
    t = jnp.where(row < POS_LOSS_ROWS, pos_t, neg_t)
    t = jnp.where((col % 16) < (CONTEXT - 1), t, 0.0)
    denom = float(NUM_WALK_ROWS * (CONTEXT - 1))
    out_ref[0, 0] = -jnp.sum(t) / denom


def _compute_loss(scores):
    scores = scores.reshape(LOSS_ROWS, 128)
    out = pl.pallas_call(
        _loss_body,
        out_specs=pl.BlockSpec(memory_space=pltpu.SMEM),
        out_shape=jax.ShapeDtypeStruct((1, 1), jnp.float32),
    )(scores)
    return out[0, 0]


def kernel(embedding_common, embedding_private_0, embedding_private_1,
           embedding_private_2, W_0, W_1, W_2, b_0, b_1, b_2, pos_rw, neg_rw):
    table = _build_embedding(embedding_common, embedding_private_0,
                             embedding_private_1, embedding_private_2,
                             W_0, W_1, W_2, b_0.reshape(1, DIM),
                             b_1.reshape(1, DIM), b_2.reshape(1, DIM))
    idx = jnp.concatenate([pos_rw.T.reshape(-1), neg_rw.T.reshape(-1)]
                          ).astype(jnp.int32)
    loc = idx % (2 * (ROW_BLOCK // 2))
    idx = (idx - loc) + (loc % (ROW_BLOCK // 2)) * 2 + loc // (ROW_BLOCK // 2)
    scores = _compute_scores(table.reshape(NUM_NODES, DIM // 2), idx)
    return _compute_loss(scores)


# speedup vs baseline: 3.0608x; 1.0232x over previous
import functools

import jax
import jax.numpy as jnp
from jax import lax
from jax.experimental import pallas as pl
from jax.experimental.pallas import tpu as pltpu
from jax.experimental.pallas import tpu_sc as plsc

NUM_NODES = 100000
DIM = 128
EPS = 1e-15
NUM_WALK_ROWS = 14080
CONTEXT = 10

ROW_BLOCK = 8000
NUM_ROW_BLOCKS = NUM_NODES // ROW_BLOCK

NW = 32
TOTAL_ROWS = 2 * NUM_WALK_ROWS
ROWS_PER_W = TOTAL_ROWS // NW
IDX_PER_W = ROWS_PER_W * CONTEXT
GROUP_ROWS = 16
IDX_PER_GROUP = GROUP_ROWS * CONTEXT
HALF_IDX = IDX_PER_GROUP // 2
N_GROUPS = ROWS_PER_W // GROUP_ROWS
PACK_WORDS = DIM // 2
D_UNROLL = 4


def _embed_body(c_ref, p0_ref, p1_ref, p2_ref, w0_ref, w1_ref, w2_ref,
                b0_ref, b1_ref, b2_ref, out_ref):
    acc = c_ref[...] + (b0_ref[...] + b1_ref[...] + b2_ref[...])
    for p_ref, w_ref in ((p0_ref, w0_ref), (p1_ref, w1_ref), (p2_ref, w2_ref)):
        acc = acc + lax.dot_general(
            p_ref[...], w_ref[...],
            dimension_numbers=(((1,), (1,)), ((), ())),
            preferred_element_type=jnp.float32)
    lo = lax.bitcast_convert_type(
        acc[:, :DIM // 2].astype(jnp.bfloat16), jnp.uint16).astype(jnp.uint32)
    hi = lax.bitcast_convert_type(
        acc[:, DIM // 2:].astype(jnp.bfloat16), jnp.uint16).astype(jnp.uint32)
    packed = lax.bitcast_convert_type(lo | (hi << 16), jnp.float32)
    out_ref[...] = jnp.concatenate(
        [packed[:ROW_BLOCK // 2], packed[ROW_BLOCK // 2:]], axis=1)


def _build_embedding(c, p0, p1, p2, w0, w1, w2, b0, b1, b2):
    row_spec = pl.BlockSpec((ROW_BLOCK, DIM), lambda i: (i, 0))
    w_spec = pl.BlockSpec((DIM, DIM), lambda i: (0, 0))
    b_spec = pl.BlockSpec((1, DIM), lambda i: (0, 0))
    out_spec = pl.BlockSpec((ROW_BLOCK // 2, DIM), lambda i: (i, 0))
    return pl.pallas_call(
        _embed_body,
        grid=(NUM_ROW_BLOCKS,),
        in_specs=[row_spec, row_spec, row_spec, row_spec,
                  w_spec, w_spec, w_spec, b_spec, b_spec, b_spec],
        out_specs=out_spec,
        out_shape=jax.ShapeDtypeStruct((NUM_NODES // 2, DIM), jnp.float32),
    )(c, p0, p1, p2, w0, w1, w2, b0, b1, b2)


def _score_body(table_hbm, idx_hbm, out_hbm, idx_v, ia0, ib0, ia1, ib1,
                rows0, rows1, scores_v, sem0, sem1):
    wid = lax.axis_index("s") * 2 + lax.axis_index("c")
    part = wid // 16
    rbase = (wid % 16) * ROWS_PER_W
    for s in range(CONTEXT):
        pltpu.sync_copy(
            idx_hbm.at[pl.ds(part * (NUM_WALK_ROWS * CONTEXT)
                             + s * NUM_WALK_ROWS + rbase, ROWS_PER_W)],
            idx_v.at[pl.ds(s * ROWS_PER_W, ROWS_PER_W)])
    lane = lax.iota(jnp.int32, 16)
    lane16 = lane * 16
    row_idx = [s * GROUP_ROWS + lane for s in range(CONTEXT)]

    def issue(g, ia, ib, rows, sem):
        for k in range(HALF_IDX // 16):
            ia[pl.ds(k * 16, 16)] = idx_v[
                pl.ds(k * ROWS_PER_W + g * GROUP_ROWS, 16)]
            ib[pl.ds(k * 16, 16)] = idx_v[
                pl.ds((k + 5) * ROWS_PER_W + g * GROUP_ROWS, 16)]
        pltpu.async_copy(table_hbm.at[ia], rows.at[pl.ds(0, HALF_IDX)], sem)
        pltpu.async_copy(
            table_hbm.at[ib], rows.at[pl.ds(HALF_IDX, HALF_IDX)], sem)

    def wait(ia, ib, rows, sem):
        pltpu.make_async_copy(
            table_hbm.at[ia], rows.at[pl.ds(0, HALF_IDX)], sem).wait()
        pltpu.make_async_copy(
            table_hbm.at[ib], rows.at[pl.ds(HALF_IDX, HALF_IDX)], sem).wait()

    def compute(g, rows):
        zero = jnp.zeros((32,), jnp.bfloat16)

        def d_body(it, accs):
            accs = list(accs)
            for u in range(D_UNROLL):
                d = it * D_UNROLL + u
                dvec = (lane + d) & (PACK_WORDS - 1)
                sw = plsc.bitcast(
                    plsc.load_gather(rows, [row_idx[0], dvec]), jnp.bfloat16)
                for j in range(CONTEXT - 1):
                    rw = plsc.bitcast(
                        plsc.load_gather(rows, [row_idx[j + 1], dvec]),
                        jnp.bfloat16)
                    accs[j] = accs[j] + sw * rw
            return tuple(accs)

        accs = lax.fori_loop(0, PACK_WORDS // D_UNROLL, d_body,
                             (zero,) * (CONTEXT - 1))
        base = g * (GROUP_ROWS * 16)
        for j in range(CONTEXT - 1):
            ua, ub = plsc.unpack(accs[j], format=plsc.PackFormat.INTERLEAVED)
            plsc.store_scatter(scores_v, [lane16 + (base + j)], ua + ub)

    issue(0, ia0, ib0, rows0, sem0)

    def pair_body(p, carry):
        ci = p * 2
        issue(ci + 1, ia1, ib1, rows1, sem1)
        wait(ia0, ib0, rows0, sem0)
        compute(ci, rows0)
        issue(ci + 2, ia0, ib0, rows0, sem0)
        wait(ia1, ib1, rows1, sem1)
        compute(ci + 1, rows1)
        return carry

    lax.fori_loop(0, (N_GROUPS - 1) // 2, pair_body, 0)
    wait(ia0, ib0, rows0, sem0)
    compute(N_GROUPS - 1, rows0)

    pltpu.sync_copy(
        scores_v,
        out_hbm.at[pl.ds(wid * ROWS_PER_W * 16, ROWS_PER_W * 16)])


def _compute_scores(table, idx):
    mesh = plsc.VectorSubcoreMesh(core_axis_name="c", subcore_axis_name="s")
    k = functools.partial(
        pl.kernel,
        out_type=jax.ShapeDtypeStruct((TOTAL_ROWS * 16,), jnp.float32),
        mesh=mesh,
        compiler_params=pltpu.CompilerParams(
            needs_layout_passes=False, use_tc_tiling_on_sc=False),
        scratch_types=[
            pltpu.VMEM((IDX_PER_W,), jnp.int32),
            pltpu.VMEM((HALF_IDX,), jnp.int32),
            pltpu.VMEM((HALF_IDX,), jnp.int32),
            pltpu.VMEM((HALF_IDX,), jnp.int32),
            pltpu.VMEM((HALF_IDX,), jnp.int32),
            pltpu.VMEM((IDX_PER_GROUP, PACK_WORDS), jnp.float32),
            pltpu.VMEM((IDX_PER_GROUP, PACK_WORDS), jnp.float32),
            pltpu.VMEM((ROWS_PER_W * 16,), jnp.float32),
            pltpu.SemaphoreType.DMA,
            pltpu.SemaphoreType.DMA,
        ],
    )(_score_body)
    return k(table, idx)


LOSS_ROWS = TOTAL_ROWS * 16 // 128
POS_LOSS_ROWS = LOSS_ROWS // 2


def _loss_body(s_ref, out_ref):
    x = s_ref[...]
    col = lax.broadcasted_iota(jnp.int32, (LOSS_ROWS, 128), 1)
    row = lax.broadcasted_iota(jnp.int32, (LOSS_ROWS, 128), 0)
    sig = 1.0 / (1.0 + jnp.exp(-x))
    pos_t = jnp.log(sig + EPS)
    neg_t = jnp.log(1.0 - sig + EPS)
    t = jnp.where(row < POS_LOSS_ROWS, pos_t, neg_t)
    t = jnp.where((col % 16) < (CONTEXT - 1), t, 0.0)
    denom = float(NUM_WALK_ROWS * (CONTEXT - 1))
    out_ref[0, 0] = -jnp.sum(t) / denom


def _compute_loss(scores):
    scores = scores.reshape(LOSS_ROWS, 128)
    out = pl.pallas_call(
        _loss_body,
        out_specs=pl.BlockSpec(memory_space=pltpu.SMEM),
        out_shape=jax.ShapeDtypeStruct((1, 1), jnp.float32),
    )(scores)
    return out[0, 0]


def kernel(embedding_common, embedding_private_0, embedding_private_1,
           embedding_private_2, W_0, W_1, W_2, b_0, b_1, b_2, pos_rw, neg_rw):
    table = _build_embedding(embedding_common, embedding_private_0,
                             embedding_private_1, embedding_private_2,
                             W_0, W_1, W_2, b_0.reshape(1, DIM),
                             b_1.reshape(1, DIM), b_2.reshape(1, DIM))
    idx = jnp.concatenate([pos_rw.T.reshape(-1), neg_rw.T.reshape(-1)]
                          ).astype(jnp.int32)
    loc = idx % (2 * (ROW_BLOCK // 2))
    idx = (idx - loc) + (loc % (ROW_BLOCK // 2)) * 2 + loc // (ROW_BLOCK // 2)
    scores = _compute_scores(table.reshape(NUM_NODES, DIM // 2), idx)
    return _compute_loss(scores)
